# baseline (speedup 1.0000x reference)
import functools

import jax
import jax.numpy as jnp
from jax import lax
from jax.experimental import pallas as pl
from jax.experimental.pallas import tpu as pltpu
from jax.experimental.pallas import tpu_sc as plsc

NUM_BLOCKS = 8
M = 2000
PM = 2048
D = 2048
NC = 2
NS = 16
NW = NC * NS
CHUNK = (NUM_BLOCKS * M) // NW


def _unpad(inp):
    mesh = plsc.VectorSubcoreMesh(core_axis_name="c", subcore_axis_name="s")

    @functools.partial(
        pl.kernel,
        mesh=mesh,
        out_type=jax.ShapeDtypeStruct((NUM_BLOCKS * M, D), jnp.float32),
    )
    def k(inp_hbm, out_hbm):
        wid = lax.axis_index("s") * NC + lax.axis_index("c")
        blk = wid // 4
        sub = wid % 4
        off = sub * 504
        src = pl.multiple_of(blk * PM + off, 8)
        dst = pl.multiple_of(blk * M + off, 8)

        @pl.when(sub < 3)
        def _copy_full():
            pltpu.sync_copy(
                inp_hbm.at[pl.ds(src, 504), :],
                out_hbm.at[pl.ds(dst, 504), :],
            )

        @pl.when(sub == 3)
        def _copy_tail():
            pltpu.sync_copy(
                inp_hbm.at[pl.ds(src, 488), :],
                out_hbm.at[pl.ds(dst, 488), :],
            )

    return k(inp)


def kernel(inp, m_splits):
    inp2d = inp.reshape(-1, inp.shape[-1])
    return _unpad(inp2d)


# speedup vs baseline: 1.0011x; 1.0011x over previous
import functools

import jax
import jax.numpy as jnp
from jax import lax
from jax.experimental import pallas as pl
from jax.experimental.pallas import tpu as pltpu
from jax.experimental.pallas import tpu_sc as plsc

NUM_BLOCKS = 8
M = 2000
PM = 2048
D = 2048
NC = 2
BLOCKS_PER_CORE = NUM_BLOCKS // NC


def _unpad(inp):
    mesh = plsc.ScalarSubcoreMesh(axis_name="c", num_cores=NC)

    @functools.partial(
        pl.kernel,
        mesh=mesh,
        out_type=jax.ShapeDtypeStruct((NUM_BLOCKS * M, D), jnp.float32),
        scratch_types=[pltpu.SemaphoreType.DMA] * BLOCKS_PER_CORE,
    )
    def k(inp_hbm, out_hbm, *sems):
        core = lax.axis_index("c")
        copies = []
        for j in range(BLOCKS_PER_CORE):
            blk = core * BLOCKS_PER_CORE + j
            src = pl.multiple_of(blk * PM, 8)
            dst = pl.multiple_of(blk * M, 8)
            cp = pltpu.make_async_copy(
                inp_hbm.at[pl.ds(src, M), :],
                out_hbm.at[pl.ds(dst, M), :],
                sems[j],
            )
            cp.start()
            copies.append(cp)
        for cp in copies:
            cp.wait()

    return k(inp)


def kernel(inp, m_splits):
    inp2d = inp.reshape(-1, inp.shape[-1])
    return _unpad(inp2d)


# speedup vs baseline: 36.5084x; 36.4671x over previous
import functools

import jax
import jax.numpy as jnp
from jax import lax
from jax.experimental import pallas as pl
from jax.experimental.pallas import tpu as pltpu
from jax.experimental.pallas import tpu_sc as plsc

NUM_BLOCKS = 8
M = 2000
PM = 2048
D = 2048
NC = 2
NS = 16
W_FULL = 504
W_TAIL = 488
C = 24
ITERS = 21
NBUF = 2


def _unpad(inp):
    mesh = plsc.VectorSubcoreMesh(core_axis_name="c", subcore_axis_name="s")

    @functools.partial(
        pl.kernel,
        mesh=mesh,
        out_type=jax.ShapeDtypeStruct((NUM_BLOCKS * M, D), jnp.float32),
        scratch_types=(
            [pltpu.VMEM((C, D), jnp.float32)] * NBUF
            + [pltpu.SemaphoreType.DMA] * (2 * NBUF)
        ),
    )
    def k(inp_hbm, out_hbm, *scr):
        bufs = scr[:NBUF]
        isems = scr[NBUF : 2 * NBUF]
        osems = scr[2 * NBUF :]
        wid = lax.axis_index("s") * NC + lax.axis_index("c")
        blk = wid // 4
        sub = wid % 4
        off = sub * W_FULL
        src0 = blk * PM + off
        dst0 = blk * M + off
        is_tail = sub == 3

        def base(i):
            b = i * C
            if i == ITERS - 1:
                b = jnp.where(is_tail, W_TAIL - C, b)
            return b

        def start_in(i):
            slot = i % NBUF
            s = pl.multiple_of(src0 + base(i), 8)
            return pltpu.async_copy(
                inp_hbm.at[pl.ds(s, C), :], bufs[slot], isems[slot]
            )

        def start_out(i):
            slot = i % NBUF
            d = pl.multiple_of(dst0 + base(i), 8)
            return pltpu.async_copy(
                bufs[slot], out_hbm.at[pl.ds(d, C), :], osems[slot]
            )

        in_h = {}
        out_h = {}
        for i in range(min(NBUF, ITERS)):
            in_h[i] = start_in(i)
        for i in range(ITERS):
            in_h[i].wait()
            out_h[i] = start_out(i)
            if i + NBUF < ITERS:
                out_h[i].wait()
                in_h[i + NBUF] = start_in(i + NBUF)
        for i in range(max(0, ITERS - NBUF), ITERS):
            out_h[i].wait()

    return k(inp)


def kernel(inp, m_splits):
    inp2d = inp.reshape(-1, inp.shape[-1])
    return _unpad(inp2d)
